# baseline (speedup 1.0000x reference)
import jax
import jax.numpy as jnp
from jax import lax
from jax.experimental import pallas as pl
from jax.experimental.pallas import tpu as pltpu
from jax.experimental.pallas import tpu_sc as plsc

N = 10000
NP = 10240
D = 128
E = 320000
T = 2
B = 1024

NW = 32
EPT = E // NW
SLP = B // NW
G = 128
CCAP = 3200
ACC = 1032
DACC = 1040
DUMP = 1024
NEG_SLOPE = 0.2


def _lrelu(x):
    return jnp.where(x > 0, x, x * NEG_SLOPE)


def _prep_body(emb_ref, wp_ref, asp_ref, adp_ref, wc_ref, asc_ref, adc_ref,
               hp_ref, hc_ref, tabs_ref, stats_ref):
    i = pl.program_id(0)
    x = emb_ref[...]
    hp = jnp.dot(x, wp_ref[...], preferred_element_type=jnp.float32)
    hc = jnp.dot(x, wc_ref[...], preferred_element_type=jnp.float32)
    hp_ref[...] = hp
    hc_ref[...] = hc
    a_sp = jnp.sum(hp * asp_ref[...], axis=1)
    a_dp = jnp.sum(hp * adp_ref[...], axis=1)
    a_sc = jnp.sum(hc * asc_ref[...], axis=1)
    a_dc = jnp.sum(hc * adc_ref[...], axis=1)
    blk = x.shape[0]
    tabs_ref[...] = jnp.concatenate(
        [a_sp.reshape(1, blk), a_dp.reshape(1, blk),
         a_sc.reshape(1, blk), a_dc.reshape(1, blk),
         jnp.zeros((4, blk), jnp.float32)], axis=0)
    r = lax.broadcasted_iota(jnp.int32, (8, 128), 0)
    cur = jnp.where(r == 0, jnp.max(a_sp),
                    jnp.where(r == 1, jnp.max(a_sc), jnp.float32(-1e30)))

    @pl.when(i == 0)
    def _():
        stats_ref[...] = cur

    @pl.when(i != 0)
    def _():
        stats_ref[...] = jnp.maximum(stats_ref[...], cur)


def _prep(emb_pad, W_p, att_src_p, att_dst_p, W_c, att_src_c, att_dst_c):
    blk = 1024
    grid = NP // blk
    return pl.pallas_call(
        _prep_body,
        grid=(grid,),
        in_specs=[
            pl.BlockSpec((blk, D), lambda i: (i, 0)),
            pl.BlockSpec((D, D), lambda i: (0, 0)),
            pl.BlockSpec((1, D), lambda i: (0, 0)),
            pl.BlockSpec((1, D), lambda i: (0, 0)),
            pl.BlockSpec((D, D), lambda i: (0, 0)),
            pl.BlockSpec((1, D), lambda i: (0, 0)),
            pl.BlockSpec((1, D), lambda i: (0, 0)),
        ],
        out_specs=[
            pl.BlockSpec((blk, D), lambda i: (i, 0)),
            pl.BlockSpec((blk, D), lambda i: (i, 0)),
            pl.BlockSpec((8, blk), lambda i: (0, i)),
            pl.BlockSpec((8, 128), lambda i: (0, 0)),
        ],
        out_shape=[
            jax.ShapeDtypeStruct((NP, D), jnp.float32),
            jax.ShapeDtypeStruct((NP, D), jnp.float32),
            jax.ShapeDtypeStruct((8, NP), jnp.float32),
            jax.ShapeDtypeStruct((8, 128), jnp.float32),
        ],
    )(emb_pad, W_p, att_src_p.reshape(1, D), att_dst_p.reshape(1, D),
      W_c, att_src_c.reshape(1, D), att_dst_c.reshape(1, D))


def _sc_body(s_hbm, esp_hbm, edp_hbm, esc_hbm, edc_hbm, esr_hbm, edr_hbm,
             hp_hbm, hc_hbm, emb_hbm, tabs_hbm, stats_hbm,
             nums_hbm, dens_hbm, qslot_hbm, embs_hbm,
             map_v, asp_v, adp_v, asc_v, adc_v, sv, esrc_v, edst_v,
             csrc_v, cdst_v, cslot_v, cw_v, rows_v, den_v, idx2_v,
             gp_v, gc_v, num_sh, sem):
    cid = lax.axis_index("c")
    sid = lax.axis_index("s")
    wid = cid * 16 + sid
    iota = lax.broadcasted_iota(jnp.int32, (16,), 0)

    pltpu.sync_copy(s_hbm, sv)
    pltpu.sync_copy(tabs_hbm.at[pl.ds(0 * NP, NP)], asp_v)
    pltpu.sync_copy(tabs_hbm.at[pl.ds(1 * NP, NP)], adp_v)
    pltpu.sync_copy(tabs_hbm.at[pl.ds(2 * NP, NP)], asc_v)
    pltpu.sync_copy(tabs_hbm.at[pl.ds(3 * NP, NP)], adc_v)
    pltpu.sync_copy(stats_hbm.at[pl.ds(0, 16)], gp_v)
    pltpu.sync_copy(stats_hbm.at[pl.ds(128, 16)], gc_v)

    def _init_map(k, carry):
        map_v[pl.ds(k * 16, 16)] = jnp.full((16,), -1, jnp.int32)
        return carry
    lax.fori_loop(0, NP // 16, _init_map, 0)

    def _fill_map(g, carry):
        s16 = sv[pl.ds(g * 16, 16)]
        b16 = g * 16 + iota
        for l in range(16):
            plsc.store_scatter(map_v, [s16], b16, mask=iota == l)
        return carry
    lax.fori_loop(0, B // 16, _fill_map, 0)

    zero16 = jnp.zeros((16,), jnp.float32)

    gmax_p = gp_v[...]
    gmax_c = gc_v[...]

    def _filter_store(src16, dst16, slot16, mask, ptr, store_dst):
        cs = plsc.cumsum(mask.astype(jnp.int32))
        pos = ptr + cs - 1
        plsc.store_scatter(csrc_v, [pos], src16, mask=mask)
        plsc.store_scatter(cslot_v, [pos], slot16, mask=mask)
        if store_dst:
            plsc.store_scatter(cdst_v, [pos], dst16, mask=mask)
        return ptr + cs[15]

    def _relate_group(g, ptr):
        src16 = esrc_v[pl.ds(g * 16, 16)]
        dst16 = edst_v[pl.ds(g * 16, 16)]
        slot16 = plsc.load_gather(map_v, [dst16])
        return _filter_store(src16, dst16, slot16, slot16 >= 0, ptr, False)

    for t in range(T):
        for rel in range(3):
            es_hbm = (esp_hbm, esc_hbm, esr_hbm)[rel]
            ed_hbm = (edp_hbm, edc_hbm, edr_hbm)[rel]
            tab_hbm = (hp_hbm, hc_hbm, emb_hbm)[rel]
            as_v = (asp_v, asc_v, None)[rel]
            ad_v = (adp_v, adc_v, None)[rel]
            gmax = (gmax_p, gmax_c, None)[rel]
            ph = t * 3 + rel
            is_gat = rel < 2

            plsc.subcore_barrier()
            if ph > 0:
                pltpu.sync_copy(
                    den_v, dens_hbm.at[pl.ds((ph - 1) * NW * DACC
                                             + wid * DACC, DACC)])

            def _zero_den(k, carry):
                den_v[pl.ds(k * 16, 16)] = zero16
                return carry
            lax.fori_loop(0, DACC // 16, _zero_den, 0)

            @pl.when(sid == 0)
            def _(ph=ph):
                if ph > 0:
                    pltpu.sync_copy(num_sh, nums_hbm.at[cid, ph - 1])

                def _zero_rows(k, carry):
                    k16 = jnp.full((16,), k, jnp.int32)
                    for j in range(D // 16):
                        plsc.store_scatter(rows_v, [k16, j * 16 + iota],
                                           zero16)
                    return carry
                lax.fori_loop(0, G, _zero_rows, 0)
                for k in range(8):
                    pltpu.sync_copy(rows_v, num_sh.at[pl.ds(k * G, G)])
                pltpu.sync_copy(rows_v.at[pl.ds(0, 8)],
                                num_sh.at[pl.ds(1024, 8)])
            plsc.subcore_barrier()

            pltpu.sync_copy(es_hbm.at[pl.ds(t * E + wid * EPT, EPT)], esrc_v)
            pltpu.sync_copy(ed_hbm.at[pl.ds(t * E + wid * EPT, EPT)], edst_v)

            scope_f = jax.named_scope(f"ph{ph}_filter")
            scope_f.__enter__()
            if is_gat:
                def _edge_group(g, ptr):
                    src16 = esrc_v[pl.ds(g * 16, 16)]
                    dst16 = edst_v[pl.ds(g * 16, 16)]
                    slot16 = plsc.load_gather(map_v, [dst16])
                    return _filter_store(src16, dst16, slot16, slot16 >= 0,
                                         ptr, True)
                ptr = lax.fori_loop(0, EPT // 16, _edge_group, jnp.int32(0))

                def _self_group(g, ptr):
                    off = wid * SLP + g * 16
                    s16 = sv[pl.ds(off, 16)]
                    slot16 = plsc.load_gather(map_v, [s16])
                    keep = slot16 == (off + iota)
                    return _filter_store(s16, s16, slot16, keep, ptr, True)
                ptr = lax.fori_loop(0, SLP // 16, _self_group, ptr)

                def _w_group(g, carry, as_v=as_v, ad_v=ad_v, gmax=gmax):
                    src16 = csrc_v[pl.ds(g * 16, 16)]
                    dst16 = cdst_v[pl.ds(g * 16, 16)]
                    src16 = jnp.minimum(jnp.maximum(src16, 0), N - 1)
                    dst16 = jnp.minimum(jnp.maximum(dst16, 0), N - 1)
                    a_s = plsc.load_gather(as_v, [src16])
                    a_d = plsc.load_gather(ad_v, [dst16])
                    al = _lrelu(a_s + a_d)
                    m = _lrelu(gmax + a_d)
                    cw_v[pl.ds(g * 16, 16)] = jnp.exp(al - m)
                    return carry
                lax.fori_loop(0, (ptr + 15) // 16, _w_group, 0)
            else:
                ptr = lax.fori_loop(0, EPT // 16, _relate_group, jnp.int32(0))

            def _pad(j, carry, ptr=ptr):
                posv = ptr + j * 16 + iota
                plsc.store_scatter(csrc_v, [posv],
                                   jnp.zeros((16,), jnp.int32))
                plsc.store_scatter(cslot_v, [posv],
                                   jnp.full((16,), DUMP, jnp.int32))
                plsc.store_scatter(cw_v, [posv], zero16)
                return carry
            lax.fori_loop(0, G // 16, _pad, 0)

            scope_f.__exit__(None, None, None)
            nchunks = (ptr + (G - 1)) // G
            scope_r = jax.named_scope(f"ph{ph}_rows")
            scope_r.__enter__()

            def _chunk(c, carry, tab_hbm=tab_hbm, ph=ph, is_gat=is_gat):
                base = c * G
                for j in range(G // 16):
                    idx2_v[0, pl.ds(j * 16, 16)] = (
                        cslot_v[pl.ds(base + j * 16, 16)])
                cp = pltpu.async_copy(tab_hbm.at[csrc_v.at[pl.ds(base, G)]],
                                      rows_v, sem)
                cp.wait()
                lane0 = iota == 0
                if is_gat:
                    def _scale(i, carry2):
                        ei16 = jnp.full((16,), base + i, jnp.int32)
                        wv = plsc.load_gather(cw_v, [ei16])
                        sl16 = plsc.load_gather(cslot_v, [ei16])
                        plsc.addupdate_scatter(den_v, [sl16], wv, mask=lane0)
                        i16 = jnp.full((16,), i, jnp.int32)
                        for j in range(D // 16):
                            cols = j * 16 + iota
                            r = plsc.load_gather(rows_v, [i16, cols])
                            plsc.store_scatter(rows_v, [i16, cols], r * wv)
                        return carry2
                    lax.fori_loop(0, G, _scale, 0)
                else:
                    one16 = jnp.ones((16,), jnp.float32)

                    def _cnt(i, carry2):
                        ei16 = jnp.full((16,), base + i, jnp.int32)
                        sl16 = plsc.load_gather(cslot_v, [ei16])
                        plsc.addupdate_scatter(den_v, [sl16], one16,
                                               mask=lane0)
                        return carry2
                    lax.fori_loop(0, G, _cnt, 0)
                pltpu.sync_copy(rows_v, num_sh.at[idx2_v.at[0]], add=True)
                return carry
            lax.fori_loop(0, nchunks, _chunk, 0)
            scope_r.__exit__(None, None, None)

    plsc.subcore_barrier()

    pltpu.sync_copy(den_v, dens_hbm.at[pl.ds(5 * NW * DACC + wid * DACC,
                                             DACC)])

    @pl.when(sid == 0)
    def _():
        pltpu.sync_copy(num_sh, nums_hbm.at[cid, 5])

    @pl.when(jnp.logical_and(cid == 0, sid == 1))
    def _():
        def _q(g, carry):
            s16 = sv[pl.ds(g * 16, 16)]
            csrc_v[pl.ds(g * 16, 16)] = plsc.load_gather(map_v, [s16])
            return carry
        lax.fori_loop(0, B // 16, _q, 0)
        pltpu.sync_copy(csrc_v.at[pl.ds(0, B)], qslot_hbm)

        def _embs(c, carry):
            cp = pltpu.async_copy(emb_hbm.at[sv.at[pl.ds(c * G, G)]],
                                  rows_v, sem)
            cp.wait()
            pltpu.sync_copy(rows_v, embs_hbm.at[pl.ds(c * G, G)])
            return carry
        lax.fori_loop(0, B // G, _embs, 0)


def _sc_edges(s, ei_p, ei_c, ei_r, hp, hc, emb_pad, tabs, stats):
    esp, edp = ei_p[:, 0, :].reshape(-1), ei_p[:, 1, :].reshape(-1)
    esc, edc = ei_c[:, 0, :].reshape(-1), ei_c[:, 1, :].reshape(-1)
    esr, edr = ei_r[:, 0, :].reshape(-1), ei_r[:, 1, :].reshape(-1)
    tabs_flat = tabs[:4].reshape(-1)
    stats_flat = stats.reshape(-1)
    mesh = plsc.VectorSubcoreMesh(core_axis_name="c", subcore_axis_name="s")
    fn = pl.kernel(
        _sc_body,
        mesh=mesh,
        compiler_params=pltpu.CompilerParams(needs_layout_passes=False),
        out_type=[
            jax.ShapeDtypeStruct((2, 6, ACC, D), jnp.float32),
            jax.ShapeDtypeStruct((6 * NW * DACC,), jnp.float32),
            jax.ShapeDtypeStruct((B,), jnp.int32),
            jax.ShapeDtypeStruct((B, D), jnp.float32),
        ],
        scratch_types=[
            pltpu.VMEM((NP,), jnp.int32),
            pltpu.VMEM((NP,), jnp.float32),
            pltpu.VMEM((NP,), jnp.float32),
            pltpu.VMEM((NP,), jnp.float32),
            pltpu.VMEM((NP,), jnp.float32),
            pltpu.VMEM((B,), jnp.int32),
            pltpu.VMEM((EPT,), jnp.int32),
            pltpu.VMEM((EPT,), jnp.int32),
            pltpu.VMEM((CCAP,), jnp.int32),
            pltpu.VMEM((CCAP,), jnp.int32),
            pltpu.VMEM((CCAP,), jnp.int32),
            pltpu.VMEM((CCAP,), jnp.float32),
            pltpu.VMEM((G, D), jnp.float32),
            pltpu.VMEM((DACC,), jnp.float32),
            pltpu.VMEM((1, G), jnp.int32),
            pltpu.VMEM((16,), jnp.float32),
            pltpu.VMEM((16,), jnp.float32),
            pltpu.VMEM_SHARED((ACC, D), jnp.float32),
            pltpu.SemaphoreType.DMA,
        ],
    )
    return fn(s, esp, edp, esc, edc, esr, edr, hp, hc, emb_pad,
              tabs_flat, stats_flat)


def _fin_body(nums_ref, dens_ref, embs_ref, qslot_ref, wl_ref, bl_ref,
              wr_ref, bp_ref, bc_ref, out_ref, res_ref):
    num = nums_ref[...]
    den = dens_ref[...]
    n_p = num[0, 0, :B] + num[1, 0, :B]
    n_c = num[0, 1, :B] + num[1, 1, :B]
    n_r = num[0, 2, :B] + num[1, 2, :B]
    d_p = jnp.sum(den[0], axis=0)[:B]
    d_c = jnp.sum(den[1], axis=0)[:B]
    d_r = jnp.sum(den[2], axis=0)[:B]
    o1 = n_p / (d_p + 1e-16)[:, None] + bp_ref[...]
    o2 = n_c / (d_c + 1e-16)[:, None] + bc_ref[...]
    mean = n_r / jnp.maximum(d_r, 1.0)[:, None]
    o3 = (jnp.dot(mean, wl_ref[...], preferred_element_type=jnp.float32)
          + bl_ref[...]
          + jnp.dot(embs_ref[...], wr_ref[...],
                    preferred_element_type=jnp.float32))
    res_ref[...] = (o1 + o2 + o3) * jnp.float32(1.0 / 3.0)

    def _gather(b, carry):
        idx = qslot_ref[0, b]
        out_ref[0, pl.ds(b, 1), :] = res_ref[pl.ds(idx, 1), :]
        return carry
    lax.fori_loop(0, B, _gather, 0)


def _finish(nums, dens, embs, qslot, W_l, b_l, W_r, b_p, b_c):
    return pl.pallas_call(
        _fin_body,
        grid=(T,),
        in_specs=[
            pl.BlockSpec((2, 3, ACC, D), lambda t: (0, t, 0, 0)),
            pl.BlockSpec((3, NW, DACC), lambda t: (t, 0, 0)),
            pl.BlockSpec((B, D), lambda t: (0, 0)),
            pl.BlockSpec((1, B), lambda t: (0, 0), memory_space=pltpu.SMEM),
            pl.BlockSpec((D, D), lambda t: (0, 0)),
            pl.BlockSpec((1, D), lambda t: (0, 0)),
            pl.BlockSpec((D, D), lambda t: (0, 0)),
            pl.BlockSpec((1, D), lambda t: (0, 0)),
            pl.BlockSpec((1, D), lambda t: (0, 0)),
        ],
        out_specs=pl.BlockSpec((1, B, D), lambda t: (t, 0, 0)),
        out_shape=jax.ShapeDtypeStruct((T, B, D), jnp.float32),
        scratch_shapes=[pltpu.VMEM((B, D), jnp.float32)],
    )(nums, dens.reshape(6, NW, DACC), embs, qslot.reshape(1, B),
      W_l, b_l.reshape(1, D), W_r, b_p.reshape(1, D), b_c.reshape(1, D))


def kernel(s, t_s, t_e, emb, W_p, att_src_p, att_dst_p, b_p,
           W_c, att_src_c, att_dst_c, b_c, W_l, b_l, W_r,
           ei_parent, ei_child, ei_relate):
    del t_s, t_e
    emb_pad = jnp.pad(emb, ((0, NP - N), (0, 0)))
    s = s.astype(jnp.int32)
    ei_p = ei_parent.astype(jnp.int32)
    ei_c = ei_child.astype(jnp.int32)
    ei_r = ei_relate.astype(jnp.int32)
    hp, hc, tabs, stats = _prep(emb_pad, W_p, att_src_p, att_dst_p,
                                W_c, att_src_c, att_dst_c)
    nums, dens, qslot, embs = _sc_edges(s, ei_p, ei_c, ei_r,
                                        hp, hc, emb_pad, tabs, stats)
    return _finish(nums, dens, embs, qslot, W_l, b_l, W_r, b_p, b_c)


# speedup vs baseline: 1.2604x; 1.2604x over previous
import jax
import jax.numpy as jnp
from jax import lax
from jax.experimental import pallas as pl
from jax.experimental.pallas import tpu as pltpu
from jax.experimental.pallas import tpu_sc as plsc

N = 10000
NP = 10240
D = 128
E = 320000
T = 2
B = 1024

NW = 32
EPT = E // NW
SLP = B // NW
G = 128
CCAP = 3200
ACC = 1032
DACC = 1040
DUMP = 1024
NEG_SLOPE = 0.2


def _lrelu(x):
    return jnp.where(x > 0, x, x * NEG_SLOPE)


def _prep_body(emb_ref, wp_ref, asp_ref, adp_ref, wc_ref, asc_ref, adc_ref,
               hp_ref, hc_ref, tabs_ref, stats_ref):
    i = pl.program_id(0)
    x = emb_ref[...]
    hp = jnp.dot(x, wp_ref[...], preferred_element_type=jnp.float32)
    hc = jnp.dot(x, wc_ref[...], preferred_element_type=jnp.float32)
    hp_ref[...] = hp
    hc_ref[...] = hc
    a_sp = jnp.sum(hp * asp_ref[...], axis=1)
    a_dp = jnp.sum(hp * adp_ref[...], axis=1)
    a_sc = jnp.sum(hc * asc_ref[...], axis=1)
    a_dc = jnp.sum(hc * adc_ref[...], axis=1)
    blk = x.shape[0]
    tabs_ref[...] = jnp.concatenate(
        [a_sp.reshape(1, blk), a_dp.reshape(1, blk),
         a_sc.reshape(1, blk), a_dc.reshape(1, blk),
         jnp.zeros((4, blk), jnp.float32)], axis=0)
    r = lax.broadcasted_iota(jnp.int32, (8, 128), 0)
    cur = jnp.where(r == 0, jnp.max(a_sp),
                    jnp.where(r == 1, jnp.max(a_sc), jnp.float32(-1e30)))

    @pl.when(i == 0)
    def _():
        stats_ref[...] = cur

    @pl.when(i != 0)
    def _():
        stats_ref[...] = jnp.maximum(stats_ref[...], cur)


def _prep(emb_pad, W_p, att_src_p, att_dst_p, W_c, att_src_c, att_dst_c):
    blk = 1024
    grid = NP // blk
    return pl.pallas_call(
        _prep_body,
        grid=(grid,),
        in_specs=[
            pl.BlockSpec((blk, D), lambda i: (i, 0)),
            pl.BlockSpec((D, D), lambda i: (0, 0)),
            pl.BlockSpec((1, D), lambda i: (0, 0)),
            pl.BlockSpec((1, D), lambda i: (0, 0)),
            pl.BlockSpec((D, D), lambda i: (0, 0)),
            pl.BlockSpec((1, D), lambda i: (0, 0)),
            pl.BlockSpec((1, D), lambda i: (0, 0)),
        ],
        out_specs=[
            pl.BlockSpec((blk, D), lambda i: (i, 0)),
            pl.BlockSpec((blk, D), lambda i: (i, 0)),
            pl.BlockSpec((8, blk), lambda i: (0, i)),
            pl.BlockSpec((8, 128), lambda i: (0, 0)),
        ],
        out_shape=[
            jax.ShapeDtypeStruct((NP, D), jnp.float32),
            jax.ShapeDtypeStruct((NP, D), jnp.float32),
            jax.ShapeDtypeStruct((8, NP), jnp.float32),
            jax.ShapeDtypeStruct((8, 128), jnp.float32),
        ],
    )(emb_pad, W_p, att_src_p.reshape(1, D), att_dst_p.reshape(1, D),
      W_c, att_src_c.reshape(1, D), att_dst_c.reshape(1, D))


def _sc_body(s_hbm, esp_hbm, edp_hbm, esc_hbm, edc_hbm, esr_hbm, edr_hbm,
             hp_hbm, hc_hbm, emb_hbm, tabs_hbm, stats_hbm,
             nums_hbm, dens_hbm, qslot_hbm, embs_hbm,
             map_v, asp_v, adp_v, asc_v, adc_v, sv, esrc_v, edst_v,
             csrc_v, cdst_v, cslot_v, cw_v, rows_a, rows_b, den_v,
             idx2_a, idx2_b, gp_v, gc_v, num_sh, sem, sga, sgb, ssa, ssb):
    cid = lax.axis_index("c")
    sid = lax.axis_index("s")
    wid = cid * 16 + sid
    iota = lax.broadcasted_iota(jnp.int32, (16,), 0)

    pltpu.sync_copy(s_hbm, sv)
    pltpu.sync_copy(tabs_hbm.at[pl.ds(0 * NP, NP)], asp_v)
    pltpu.sync_copy(tabs_hbm.at[pl.ds(1 * NP, NP)], adp_v)
    pltpu.sync_copy(tabs_hbm.at[pl.ds(2 * NP, NP)], asc_v)
    pltpu.sync_copy(tabs_hbm.at[pl.ds(3 * NP, NP)], adc_v)
    pltpu.sync_copy(stats_hbm.at[pl.ds(0, 16)], gp_v)
    pltpu.sync_copy(stats_hbm.at[pl.ds(128, 16)], gc_v)

    def _init_map(k, carry):
        map_v[pl.ds(k * 16, 16)] = jnp.full((16,), -1, jnp.int32)
        return carry
    lax.fori_loop(0, NP // 16, _init_map, 0)

    def _fill_map(g, carry):
        s16 = sv[pl.ds(g * 16, 16)]
        b16 = g * 16 + iota
        for l in range(16):
            plsc.store_scatter(map_v, [s16], b16, mask=iota == l)
        return carry
    lax.fori_loop(0, B // 16, _fill_map, 0)

    zero16 = jnp.zeros((16,), jnp.float32)

    gmax_p = gp_v[...]
    gmax_c = gc_v[...]

    def _filter_store(src16, dst16, slot16, mask, ptr, store_dst):
        cs = plsc.cumsum(mask.astype(jnp.int32))
        pos = ptr + cs - 1
        plsc.store_scatter(csrc_v, [pos], src16, mask=mask)
        plsc.store_scatter(cslot_v, [pos], slot16, mask=mask)
        if store_dst:
            plsc.store_scatter(cdst_v, [pos], dst16, mask=mask)
        return ptr + cs[15]

    def _relate_group(g, ptr):
        src16 = esrc_v[pl.ds(g * 16, 16)]
        dst16 = edst_v[pl.ds(g * 16, 16)]
        slot16 = plsc.load_gather(map_v, [dst16])
        return _filter_store(src16, dst16, slot16, slot16 >= 0, ptr, False)

    for t in range(T):
        for rel in range(3):
            es_hbm = (esp_hbm, esc_hbm, esr_hbm)[rel]
            ed_hbm = (edp_hbm, edc_hbm, edr_hbm)[rel]
            tab_hbm = (hp_hbm, hc_hbm, emb_hbm)[rel]
            as_v = (asp_v, asc_v, None)[rel]
            ad_v = (adp_v, adc_v, None)[rel]
            gmax = (gmax_p, gmax_c, None)[rel]
            ph = t * 3 + rel
            is_gat = rel < 2

            plsc.subcore_barrier()
            if ph > 0:
                pltpu.sync_copy(
                    den_v, dens_hbm.at[pl.ds((ph - 1) * NW * DACC
                                             + wid * DACC, DACC)])

            def _zero_den(k, carry):
                den_v[pl.ds(k * 16, 16)] = zero16
                return carry
            lax.fori_loop(0, DACC // 16, _zero_den, 0)

            @pl.when(sid == 0)
            def _(ph=ph):
                if ph > 0:
                    pltpu.sync_copy(num_sh, nums_hbm.at[cid, ph - 1])

                def _zero_rows(k, carry):
                    k16 = jnp.full((16,), k, jnp.int32)
                    for j in range(D // 16):
                        plsc.store_scatter(rows_a, [k16, j * 16 + iota],
                                           zero16)
                    return carry
                lax.fori_loop(0, G, _zero_rows, 0)
                for k in range(8):
                    pltpu.sync_copy(rows_a, num_sh.at[pl.ds(k * G, G)])
                pltpu.sync_copy(rows_a.at[pl.ds(0, 8)],
                                num_sh.at[pl.ds(1024, 8)])
            plsc.subcore_barrier()

            pltpu.sync_copy(es_hbm.at[pl.ds(t * E + wid * EPT, EPT)], esrc_v)
            pltpu.sync_copy(ed_hbm.at[pl.ds(t * E + wid * EPT, EPT)], edst_v)

            if is_gat:
                def _edge_group(g, ptr):
                    src16 = esrc_v[pl.ds(g * 16, 16)]
                    dst16 = edst_v[pl.ds(g * 16, 16)]
                    slot16 = plsc.load_gather(map_v, [dst16])
                    return _filter_store(src16, dst16, slot16, slot16 >= 0,
                                         ptr, True)
                ptr = lax.fori_loop(0, EPT // 16, _edge_group, jnp.int32(0))

                def _self_group(g, ptr):
                    off = wid * SLP + g * 16
                    s16 = sv[pl.ds(off, 16)]
                    slot16 = plsc.load_gather(map_v, [s16])
                    keep = slot16 == (off + iota)
                    return _filter_store(s16, s16, slot16, keep, ptr, True)
                ptr = lax.fori_loop(0, SLP // 16, _self_group, ptr)

                def _w_group(g, carry, as_v=as_v, ad_v=ad_v, gmax=gmax):
                    src16 = csrc_v[pl.ds(g * 16, 16)]
                    dst16 = cdst_v[pl.ds(g * 16, 16)]
                    src16 = jnp.minimum(jnp.maximum(src16, 0), N - 1)
                    dst16 = jnp.minimum(jnp.maximum(dst16, 0), N - 1)
                    a_s = plsc.load_gather(as_v, [src16])
                    a_d = plsc.load_gather(ad_v, [dst16])
                    al = _lrelu(a_s + a_d)
                    m = _lrelu(gmax + a_d)
                    cw_v[pl.ds(g * 16, 16)] = jnp.exp(al - m)
                    return carry
                lax.fori_loop(0, (ptr + 15) // 16, _w_group, 0)
            else:
                ptr = lax.fori_loop(0, EPT // 16, _relate_group, jnp.int32(0))

            def _pad(j, carry, ptr=ptr):
                posv = ptr + j * 16 + iota
                plsc.store_scatter(csrc_v, [posv],
                                   jnp.zeros((16,), jnp.int32))
                plsc.store_scatter(cslot_v, [posv],
                                   jnp.full((16,), DUMP, jnp.int32))
                plsc.store_scatter(cw_v, [posv], zero16)
                return carry
            lax.fori_loop(0, G // 16, _pad, 0)

            nchunks = (ptr + (G - 1)) // G

            def _fill_idx2(idx2, base):
                for j in range(G // 16):
                    idx2[0, pl.ds(j * 16, 16)] = (
                        cslot_v[pl.ds(base + j * 16, 16)])

            lane0 = iota == 0
            one16 = jnp.ones((16,), jnp.float32)

            def _scale_rows(rows, base):
                @plsc.parallel_loop(0, G, unroll=4)
                def _(i):
                    ei16 = jnp.full((16,), base + i, jnp.int32)
                    wv = plsc.load_gather(cw_v, [ei16])
                    sl16 = plsc.load_gather(cslot_v, [ei16])
                    plsc.addupdate_scatter(den_v, [sl16], wv, mask=lane0)
                    i16 = jnp.full((16,), i, jnp.int32)
                    for j in range(D // 16):
                        cols = j * 16 + iota
                        r = plsc.load_gather(rows, [i16, cols])
                        plsc.store_scatter(rows, [i16, cols], r * wv)

            def _den_rows(base):
                @plsc.parallel_loop(0, G, unroll=4)
                def _(i):
                    ei16 = jnp.full((16,), base + i, jnp.int32)
                    sl16 = plsc.load_gather(cslot_v, [ei16])
                    plsc.addupdate_scatter(den_v, [sl16], one16, mask=lane0)

            npairs = (nchunks + 1) // 2

            def _pair(p, carry, tab_hbm=tab_hbm, is_gat=is_gat,
                      nchunks=nchunks):
                baseA = (2 * p) * G
                baseB = (2 * p + 1) * G
                has_b = (2 * p + 1) < nchunks
                SG = G // 4
                for q in range(4):
                    pltpu.async_copy(
                        tab_hbm.at[csrc_v.at[pl.ds(baseA + q * SG, SG)]],
                        rows_a.at[pl.ds(q * SG, SG)], sga)

                @pl.when(has_b)
                def _():
                    for q in range(4):
                        pltpu.async_copy(
                            tab_hbm.at[csrc_v.at[pl.ds(baseB + q * SG, SG)]],
                            rows_b.at[pl.ds(q * SG, SG)], sgb)

                _fill_idx2(idx2_a, baseA)
                for q in range(4):
                    pltpu.make_async_copy(
                        tab_hbm.at[csrc_v.at[pl.ds(baseA + q * SG, SG)]],
                        rows_a.at[pl.ds(q * SG, SG)], sga).wait()
                if is_gat:
                    _scale_rows(rows_a, baseA)
                else:
                    _den_rows(baseA)
                hsa = pltpu.async_copy(rows_a, num_sh.at[idx2_a.at[0]], ssa,
                                       add=True)

                @pl.when(has_b)
                def _():
                    _fill_idx2(idx2_b, baseB)
                    for q in range(4):
                        pltpu.make_async_copy(
                            tab_hbm.at[csrc_v.at[pl.ds(baseB + q * SG, SG)]],
                            rows_b.at[pl.ds(q * SG, SG)], sgb).wait()
                    if is_gat:
                        _scale_rows(rows_b, baseB)
                    else:
                        _den_rows(baseB)
                    pltpu.async_copy(rows_b, num_sh.at[idx2_b.at[0]], ssb,
                                     add=True)

                hsa.wait()

                @pl.when(has_b)
                def _():
                    pltpu.make_async_copy(rows_b, num_sh.at[idx2_b.at[0]],
                                          ssb).wait()
                return carry
            lax.fori_loop(0, npairs, _pair, 0)

    plsc.subcore_barrier()

    pltpu.sync_copy(den_v, dens_hbm.at[pl.ds(5 * NW * DACC + wid * DACC,
                                             DACC)])

    @pl.when(sid == 0)
    def _():
        pltpu.sync_copy(num_sh, nums_hbm.at[cid, 5])

    @pl.when(jnp.logical_and(cid == 0, sid == 1))
    def _():
        def _q(g, carry):
            s16 = sv[pl.ds(g * 16, 16)]
            csrc_v[pl.ds(g * 16, 16)] = plsc.load_gather(map_v, [s16])
            return carry
        lax.fori_loop(0, B // 16, _q, 0)
        pltpu.sync_copy(csrc_v.at[pl.ds(0, B)], qslot_hbm)

        def _embs(c, carry):
            cp = pltpu.async_copy(emb_hbm.at[sv.at[pl.ds(c * G, G)]],
                                  rows_a, sem)
            cp.wait()
            pltpu.sync_copy(rows_a, embs_hbm.at[pl.ds(c * G, G)])
            return carry
        lax.fori_loop(0, B // G, _embs, 0)


def _sc_edges(s, ei_p, ei_c, ei_r, hp, hc, emb_pad, tabs, stats):
    esp, edp = ei_p[:, 0, :].reshape(-1), ei_p[:, 1, :].reshape(-1)
    esc, edc = ei_c[:, 0, :].reshape(-1), ei_c[:, 1, :].reshape(-1)
    esr, edr = ei_r[:, 0, :].reshape(-1), ei_r[:, 1, :].reshape(-1)
    tabs_flat = tabs[:4].reshape(-1)
    stats_flat = stats.reshape(-1)
    mesh = plsc.VectorSubcoreMesh(core_axis_name="c", subcore_axis_name="s")
    fn = pl.kernel(
        _sc_body,
        mesh=mesh,
        compiler_params=pltpu.CompilerParams(needs_layout_passes=False),
        out_type=[
            jax.ShapeDtypeStruct((2, 6, ACC, D), jnp.float32),
            jax.ShapeDtypeStruct((6 * NW * DACC,), jnp.float32),
            jax.ShapeDtypeStruct((B,), jnp.int32),
            jax.ShapeDtypeStruct((B, D), jnp.float32),
        ],
        scratch_types=[
            pltpu.VMEM((NP,), jnp.int32),
            pltpu.VMEM((NP,), jnp.float32),
            pltpu.VMEM((NP,), jnp.float32),
            pltpu.VMEM((NP,), jnp.float32),
            pltpu.VMEM((NP,), jnp.float32),
            pltpu.VMEM((B,), jnp.int32),
            pltpu.VMEM((EPT,), jnp.int32),
            pltpu.VMEM((EPT,), jnp.int32),
            pltpu.VMEM((CCAP,), jnp.int32),
            pltpu.VMEM((CCAP,), jnp.int32),
            pltpu.VMEM((CCAP,), jnp.int32),
            pltpu.VMEM((CCAP,), jnp.float32),
            pltpu.VMEM((G, D), jnp.float32),
            pltpu.VMEM((G, D), jnp.float32),
            pltpu.VMEM((DACC,), jnp.float32),
            pltpu.VMEM((1, G), jnp.int32),
            pltpu.VMEM((1, G), jnp.int32),
            pltpu.VMEM((16,), jnp.float32),
            pltpu.VMEM((16,), jnp.float32),
            pltpu.VMEM_SHARED((ACC, D), jnp.float32),
            pltpu.SemaphoreType.DMA,
            pltpu.SemaphoreType.DMA,
            pltpu.SemaphoreType.DMA,
            pltpu.SemaphoreType.DMA,
            pltpu.SemaphoreType.DMA,
        ],
    )
    return fn(s, esp, edp, esc, edc, esr, edr, hp, hc, emb_pad,
              tabs_flat, stats_flat)


def _fin_body(nums_ref, dens_ref, embs_ref, qslot_ref, wl_ref, bl_ref,
              wr_ref, bp_ref, bc_ref, out_ref, res_ref):
    num = nums_ref[...]
    den = dens_ref[...]
    n_p = num[0, 0, :B] + num[1, 0, :B]
    n_c = num[0, 1, :B] + num[1, 1, :B]
    n_r = num[0, 2, :B] + num[1, 2, :B]
    d_p = jnp.sum(den[0], axis=0)[:B]
    d_c = jnp.sum(den[1], axis=0)[:B]
    d_r = jnp.sum(den[2], axis=0)[:B]
    o1 = n_p / (d_p + 1e-16)[:, None] + bp_ref[...]
    o2 = n_c / (d_c + 1e-16)[:, None] + bc_ref[...]
    mean = n_r / jnp.maximum(d_r, 1.0)[:, None]
    o3 = (jnp.dot(mean, wl_ref[...], preferred_element_type=jnp.float32)
          + bl_ref[...]
          + jnp.dot(embs_ref[...], wr_ref[...],
                    preferred_element_type=jnp.float32))
    res_ref[...] = (o1 + o2 + o3) * jnp.float32(1.0 / 3.0)

    def _gather(b, carry):
        idx = qslot_ref[0, b]
        out_ref[0, pl.ds(b, 1), :] = res_ref[pl.ds(idx, 1), :]
        return carry
    lax.fori_loop(0, B, _gather, 0)


def _finish(nums, dens, embs, qslot, W_l, b_l, W_r, b_p, b_c):
    return pl.pallas_call(
        _fin_body,
        grid=(T,),
        in_specs=[
            pl.BlockSpec((2, 3, ACC, D), lambda t: (0, t, 0, 0)),
            pl.BlockSpec((3, NW, DACC), lambda t: (t, 0, 0)),
            pl.BlockSpec((B, D), lambda t: (0, 0)),
            pl.BlockSpec((1, B), lambda t: (0, 0), memory_space=pltpu.SMEM),
            pl.BlockSpec((D, D), lambda t: (0, 0)),
            pl.BlockSpec((1, D), lambda t: (0, 0)),
            pl.BlockSpec((D, D), lambda t: (0, 0)),
            pl.BlockSpec((1, D), lambda t: (0, 0)),
            pl.BlockSpec((1, D), lambda t: (0, 0)),
        ],
        out_specs=pl.BlockSpec((1, B, D), lambda t: (t, 0, 0)),
        out_shape=jax.ShapeDtypeStruct((T, B, D), jnp.float32),
        scratch_shapes=[pltpu.VMEM((B, D), jnp.float32)],
    )(nums, dens.reshape(6, NW, DACC), embs, qslot.reshape(1, B),
      W_l, b_l.reshape(1, D), W_r, b_p.reshape(1, D), b_c.reshape(1, D))


def kernel(s, t_s, t_e, emb, W_p, att_src_p, att_dst_p, b_p,
           W_c, att_src_c, att_dst_c, b_c, W_l, b_l, W_r,
           ei_parent, ei_child, ei_relate):
    del t_s, t_e
    emb_pad = jnp.pad(emb, ((0, NP - N), (0, 0)))
    s = s.astype(jnp.int32)
    ei_p = ei_parent.astype(jnp.int32)
    ei_c = ei_child.astype(jnp.int32)
    ei_r = ei_relate.astype(jnp.int32)
    hp, hc, tabs, stats = _prep(emb_pad, W_p, att_src_p, att_dst_p,
                                W_c, att_src_c, att_dst_c)
    nums, dens, qslot, embs = _sc_edges(s, ei_p, ei_c, ei_r,
                                        hp, hc, emb_pad, tabs, stats)
    return _finish(nums, dens, embs, qslot, W_l, b_l, W_r, b_p, b_c)


# speedup vs baseline: 1.2786x; 1.0144x over previous
import jax
import jax.numpy as jnp
from jax import lax
from jax.experimental import pallas as pl
from jax.experimental.pallas import tpu as pltpu
from jax.experimental.pallas import tpu_sc as plsc

N = 10000
NP = 10240
D = 128
E = 320000
T = 2
B = 1024

NW = 32
EPT = E // NW
SLP = B // NW
G = 128
CCAP = 3200
ACC = 1032
DACC = 1040
DUMP = 1024
NEG_SLOPE = 0.2


def _lrelu(x):
    return jnp.where(x > 0, x, x * NEG_SLOPE)


def _prep_body(emb_ref, wp_ref, asp_ref, adp_ref, wc_ref, asc_ref, adc_ref,
               hp_ref, hc_ref, tabs_ref, stats_ref):
    i = pl.program_id(0)
    x = emb_ref[...]
    hp = jnp.dot(x, wp_ref[...], preferred_element_type=jnp.float32)
    hc = jnp.dot(x, wc_ref[...], preferred_element_type=jnp.float32)
    hp_ref[...] = hp
    hc_ref[...] = hc
    a_sp = jnp.sum(hp * asp_ref[...], axis=1)
    a_dp = jnp.sum(hp * adp_ref[...], axis=1)
    a_sc = jnp.sum(hc * asc_ref[...], axis=1)
    a_dc = jnp.sum(hc * adc_ref[...], axis=1)
    blk = x.shape[0]
    tabs_ref[...] = jnp.concatenate(
        [a_sp.reshape(1, blk), a_dp.reshape(1, blk),
         a_sc.reshape(1, blk), a_dc.reshape(1, blk),
         jnp.zeros((4, blk), jnp.float32)], axis=0)
    r = lax.broadcasted_iota(jnp.int32, (8, 128), 0)
    cur = jnp.where(r == 0, jnp.max(a_sp),
                    jnp.where(r == 1, jnp.max(a_sc), jnp.float32(-1e30)))

    @pl.when(i == 0)
    def _():
        stats_ref[...] = cur

    @pl.when(i != 0)
    def _():
        stats_ref[...] = jnp.maximum(stats_ref[...], cur)


def _prep(emb_pad, W_p, att_src_p, att_dst_p, W_c, att_src_c, att_dst_c):
    blk = 1024
    grid = NP // blk
    return pl.pallas_call(
        _prep_body,
        grid=(grid,),
        in_specs=[
            pl.BlockSpec((blk, D), lambda i: (i, 0)),
            pl.BlockSpec((D, D), lambda i: (0, 0)),
            pl.BlockSpec((1, D), lambda i: (0, 0)),
            pl.BlockSpec((1, D), lambda i: (0, 0)),
            pl.BlockSpec((D, D), lambda i: (0, 0)),
            pl.BlockSpec((1, D), lambda i: (0, 0)),
            pl.BlockSpec((1, D), lambda i: (0, 0)),
        ],
        out_specs=[
            pl.BlockSpec((blk, D), lambda i: (i, 0)),
            pl.BlockSpec((blk, D), lambda i: (i, 0)),
            pl.BlockSpec((8, blk), lambda i: (0, i)),
            pl.BlockSpec((8, 128), lambda i: (0, 0)),
        ],
        out_shape=[
            jax.ShapeDtypeStruct((NP, D), jnp.float32),
            jax.ShapeDtypeStruct((NP, D), jnp.float32),
            jax.ShapeDtypeStruct((8, NP), jnp.float32),
            jax.ShapeDtypeStruct((8, 128), jnp.float32),
        ],
    )(emb_pad, W_p, att_src_p.reshape(1, D), att_dst_p.reshape(1, D),
      W_c, att_src_c.reshape(1, D), att_dst_c.reshape(1, D))


def _sc_body(s_hbm, esp_hbm, edp_hbm, esc_hbm, edc_hbm, esr_hbm, edr_hbm,
             hp_hbm, hc_hbm, emb_hbm, tabs_hbm, stats_hbm,
             nums_hbm, dens_hbm, qslot_hbm, embs_hbm,
             map_v, asp_v, adp_v, asc_v, adc_v, sv, esrc_v, edst_v,
             csrc_v, cdst_v, cslot_v, cw_v, rows_a, rows_b, den_v,
             idx2_a, idx2_b, gp_v, gc_v, num_sh, sem, sga, sgb, ssa, ssb):
    cid = lax.axis_index("c")
    sid = lax.axis_index("s")
    wid = cid * 16 + sid
    iota = lax.broadcasted_iota(jnp.int32, (16,), 0)

    pltpu.sync_copy(s_hbm, sv)
    pltpu.sync_copy(tabs_hbm.at[pl.ds(0 * NP, NP)], asp_v)
    pltpu.sync_copy(tabs_hbm.at[pl.ds(1 * NP, NP)], adp_v)
    pltpu.sync_copy(tabs_hbm.at[pl.ds(2 * NP, NP)], asc_v)
    pltpu.sync_copy(tabs_hbm.at[pl.ds(3 * NP, NP)], adc_v)
    pltpu.sync_copy(stats_hbm.at[pl.ds(0, 16)], gp_v)
    pltpu.sync_copy(stats_hbm.at[pl.ds(128, 16)], gc_v)

    def _init_map(k, carry):
        map_v[pl.ds(k * 16, 16)] = jnp.full((16,), -1, jnp.int32)
        return carry
    lax.fori_loop(0, NP // 16, _init_map, 0)

    def _fill_map(g, carry):
        s16 = sv[pl.ds(g * 16, 16)]
        b16 = g * 16 + iota
        for l in range(16):
            plsc.store_scatter(map_v, [s16], b16, mask=iota == l)
        return carry
    lax.fori_loop(0, B // 16, _fill_map, 0)

    zero16 = jnp.zeros((16,), jnp.float32)

    gmax_p = gp_v[...]
    gmax_c = gc_v[...]

    def _filter_store(src16, dst16, slot16, mask, ptr, store_dst):
        cs = plsc.cumsum(mask.astype(jnp.int32))
        pos = ptr + cs - 1
        plsc.store_scatter(csrc_v, [pos], src16, mask=mask)
        plsc.store_scatter(cslot_v, [pos], slot16, mask=mask)
        if store_dst:
            plsc.store_scatter(cdst_v, [pos], dst16, mask=mask)
        return ptr + cs[15]

    def _relate_group(g, ptr):
        src16 = esrc_v[pl.ds(g * 16, 16)]
        dst16 = edst_v[pl.ds(g * 16, 16)]
        slot16 = plsc.load_gather(map_v, [dst16])
        return _filter_store(src16, dst16, slot16, slot16 >= 0, ptr, False)

    for t in range(T):
        for rel in range(3):
            es_hbm = (esp_hbm, esc_hbm, esr_hbm)[rel]
            ed_hbm = (edp_hbm, edc_hbm, edr_hbm)[rel]
            tab_hbm = (hp_hbm, hc_hbm, emb_hbm)[rel]
            as_v = (asp_v, asc_v, None)[rel]
            ad_v = (adp_v, adc_v, None)[rel]
            gmax = (gmax_p, gmax_c, None)[rel]
            ph = t * 3 + rel
            is_gat = rel < 2

            plsc.subcore_barrier()
            if ph > 0:
                pltpu.sync_copy(
                    den_v, dens_hbm.at[pl.ds((ph - 1) * NW * DACC
                                             + wid * DACC, DACC)])

            def _zero_den(k, carry):
                den_v[pl.ds(k * 16, 16)] = zero16
                return carry
            lax.fori_loop(0, DACC // 16, _zero_den, 0)

            @pl.when(sid == 0)
            def _(ph=ph):
                if ph > 0:
                    pltpu.sync_copy(num_sh, nums_hbm.at[cid, ph - 1])

                def _zero_rows(k, carry):
                    k16 = jnp.full((16,), k, jnp.int32)
                    for j in range(D // 16):
                        plsc.store_scatter(rows_a, [k16, j * 16 + iota],
                                           zero16)
                    return carry
                lax.fori_loop(0, G, _zero_rows, 0)
                for k in range(8):
                    pltpu.sync_copy(rows_a, num_sh.at[pl.ds(k * G, G)])
                pltpu.sync_copy(rows_a.at[pl.ds(0, 8)],
                                num_sh.at[pl.ds(1024, 8)])
            plsc.subcore_barrier()

            pltpu.sync_copy(es_hbm.at[pl.ds(t * E + wid * EPT, EPT)], esrc_v)
            pltpu.sync_copy(ed_hbm.at[pl.ds(t * E + wid * EPT, EPT)], edst_v)

            if is_gat:
                def _edge_group(g, ptr):
                    src16 = esrc_v[pl.ds(g * 16, 16)]
                    dst16 = edst_v[pl.ds(g * 16, 16)]
                    slot16 = plsc.load_gather(map_v, [dst16])
                    return _filter_store(src16, dst16, slot16, slot16 >= 0,
                                         ptr, True)
                ptr = lax.fori_loop(0, EPT // 16, _edge_group, jnp.int32(0))

                def _self_group(g, ptr):
                    off = wid * SLP + g * 16
                    s16 = sv[pl.ds(off, 16)]
                    slot16 = plsc.load_gather(map_v, [s16])
                    keep = slot16 == (off + iota)
                    return _filter_store(s16, s16, slot16, keep, ptr, True)
                ptr = lax.fori_loop(0, SLP // 16, _self_group, ptr)

                def _w_group(g, carry, as_v=as_v, ad_v=ad_v, gmax=gmax):
                    src16 = csrc_v[pl.ds(g * 16, 16)]
                    dst16 = cdst_v[pl.ds(g * 16, 16)]
                    src16 = jnp.minimum(jnp.maximum(src16, 0), N - 1)
                    dst16 = jnp.minimum(jnp.maximum(dst16, 0), N - 1)
                    a_s = plsc.load_gather(as_v, [src16])
                    a_d = plsc.load_gather(ad_v, [dst16])
                    al = _lrelu(a_s + a_d)
                    m = _lrelu(gmax + a_d)
                    cw_v[pl.ds(g * 16, 16)] = jnp.exp(al - m)
                    return carry
                lax.fori_loop(0, (ptr + 15) // 16, _w_group, 0)
            else:
                ptr = lax.fori_loop(0, EPT // 16, _relate_group, jnp.int32(0))

            def _pad(j, carry, ptr=ptr):
                posv = ptr + j * 16 + iota
                plsc.store_scatter(csrc_v, [posv],
                                   jnp.zeros((16,), jnp.int32))
                plsc.store_scatter(cslot_v, [posv],
                                   jnp.full((16,), DUMP, jnp.int32))
                plsc.store_scatter(cw_v, [posv], zero16)
                return carry
            lax.fori_loop(0, G // 16, _pad, 0)

            nchunks = (ptr + (G - 1)) // G

            def _fill_idx2(idx2, base):
                for j in range(G // 16):
                    idx2[0, pl.ds(j * 16, 16)] = (
                        cslot_v[pl.ds(base + j * 16, 16)])

            lane0 = iota == 0
            one16 = jnp.ones((16,), jnp.float32)

            def _scale_rows(rows, base):
                @plsc.parallel_loop(0, G, unroll=4)
                def _(i):
                    ei16 = jnp.full((16,), base + i, jnp.int32)
                    wv = plsc.load_gather(cw_v, [ei16])
                    sl16 = plsc.load_gather(cslot_v, [ei16])
                    plsc.addupdate_scatter(den_v, [sl16], wv, mask=lane0)
                    i16 = jnp.full((16,), i, jnp.int32)
                    for j in range(D // 16):
                        cols = j * 16 + iota
                        r = plsc.load_gather(rows, [i16, cols])
                        plsc.store_scatter(rows, [i16, cols], r * wv)

            def _den_rows(base):
                @plsc.parallel_loop(0, G, unroll=4)
                def _(i):
                    ei16 = jnp.full((16,), base + i, jnp.int32)
                    sl16 = plsc.load_gather(cslot_v, [ei16])
                    plsc.addupdate_scatter(den_v, [sl16], one16, mask=lane0)

            npairs = (nchunks + 1) // 2
            SG = G // 4

            def _gA(c, tab_hbm=tab_hbm):
                for q in range(4):
                    pltpu.async_copy(
                        tab_hbm.at[csrc_v.at[pl.ds(c * G + q * SG, SG)]],
                        rows_a.at[pl.ds(q * SG, SG)], sga)

            def _wA(c, tab_hbm=tab_hbm):
                for q in range(4):
                    pltpu.make_async_copy(
                        tab_hbm.at[csrc_v.at[pl.ds(c * G + q * SG, SG)]],
                        rows_a.at[pl.ds(q * SG, SG)], sga).wait()

            def _gB(c, tab_hbm=tab_hbm):
                for q in range(4):
                    pltpu.async_copy(
                        tab_hbm.at[csrc_v.at[pl.ds(c * G + q * SG, SG)]],
                        rows_b.at[pl.ds(q * SG, SG)], sgb)

            def _wB(c, tab_hbm=tab_hbm):
                for q in range(4):
                    pltpu.make_async_copy(
                        tab_hbm.at[csrc_v.at[pl.ds(c * G + q * SG, SG)]],
                        rows_b.at[pl.ds(q * SG, SG)], sgb).wait()

            @pl.when(nchunks > 0)
            def _():
                _gA(0)

            @pl.when(nchunks > 1)
            def _():
                _gB(1)

            def _pair(p, carry, is_gat=is_gat, nchunks=nchunks):
                cA = 2 * p
                cB = 2 * p + 1
                has_b = cB < nchunks
                baseA = cA * G
                baseB = cB * G
                _fill_idx2(idx2_a, baseA)
                _wA(cA)
                if is_gat:
                    _scale_rows(rows_a, baseA)
                else:
                    _den_rows(baseA)
                hsa = pltpu.async_copy(rows_a, num_sh.at[idx2_a.at[0]], ssa,
                                       add=True)

                @pl.when(has_b)
                def _():
                    _fill_idx2(idx2_b, baseB)
                    _wB(cB)
                    if is_gat:
                        _scale_rows(rows_b, baseB)
                    else:
                        _den_rows(baseB)
                    pltpu.async_copy(rows_b, num_sh.at[idx2_b.at[0]], ssb,
                                     add=True)

                hsa.wait()

                @pl.when(cA + 2 < nchunks)
                def _():
                    _gA(cA + 2)

                @pl.when(has_b)
                def _():
                    pltpu.make_async_copy(rows_b, num_sh.at[idx2_b.at[0]],
                                          ssb).wait()

                @pl.when(cB + 2 < nchunks)
                def _():
                    _gB(cB + 2)
                return carry
            lax.fori_loop(0, npairs, _pair, 0)

    plsc.subcore_barrier()

    pltpu.sync_copy(den_v, dens_hbm.at[pl.ds(5 * NW * DACC + wid * DACC,
                                             DACC)])

    @pl.when(sid == 0)
    def _():
        pltpu.sync_copy(num_sh, nums_hbm.at[cid, 5])

    @pl.when(jnp.logical_and(cid == 0, sid == 1))
    def _():
        def _q(g, carry):
            s16 = sv[pl.ds(g * 16, 16)]
            csrc_v[pl.ds(g * 16, 16)] = plsc.load_gather(map_v, [s16])
            return carry
        lax.fori_loop(0, B // 16, _q, 0)
        pltpu.sync_copy(csrc_v.at[pl.ds(0, B)], qslot_hbm)

        def _embs(c, carry):
            cp = pltpu.async_copy(emb_hbm.at[sv.at[pl.ds(c * G, G)]],
                                  rows_a, sem)
            cp.wait()
            pltpu.sync_copy(rows_a, embs_hbm.at[pl.ds(c * G, G)])
            return carry
        lax.fori_loop(0, B // G, _embs, 0)


def _sc_edges(s, ei_p, ei_c, ei_r, hp, hc, emb_pad, tabs, stats):
    esp, edp = ei_p[:, 0, :].reshape(-1), ei_p[:, 1, :].reshape(-1)
    esc, edc = ei_c[:, 0, :].reshape(-1), ei_c[:, 1, :].reshape(-1)
    esr, edr = ei_r[:, 0, :].reshape(-1), ei_r[:, 1, :].reshape(-1)
    tabs_flat = tabs[:4].reshape(-1)
    stats_flat = stats.reshape(-1)
    mesh = plsc.VectorSubcoreMesh(core_axis_name="c", subcore_axis_name="s")
    fn = pl.kernel(
        _sc_body,
        mesh=mesh,
        compiler_params=pltpu.CompilerParams(needs_layout_passes=False),
        out_type=[
            jax.ShapeDtypeStruct((2, 6, ACC, D), jnp.float32),
            jax.ShapeDtypeStruct((6 * NW * DACC,), jnp.float32),
            jax.ShapeDtypeStruct((B,), jnp.int32),
            jax.ShapeDtypeStruct((B, D), jnp.float32),
        ],
        scratch_types=[
            pltpu.VMEM((NP,), jnp.int32),
            pltpu.VMEM((NP,), jnp.float32),
            pltpu.VMEM((NP,), jnp.float32),
            pltpu.VMEM((NP,), jnp.float32),
            pltpu.VMEM((NP,), jnp.float32),
            pltpu.VMEM((B,), jnp.int32),
            pltpu.VMEM((EPT,), jnp.int32),
            pltpu.VMEM((EPT,), jnp.int32),
            pltpu.VMEM((CCAP,), jnp.int32),
            pltpu.VMEM((CCAP,), jnp.int32),
            pltpu.VMEM((CCAP,), jnp.int32),
            pltpu.VMEM((CCAP,), jnp.float32),
            pltpu.VMEM((G, D), jnp.float32),
            pltpu.VMEM((G, D), jnp.float32),
            pltpu.VMEM((DACC,), jnp.float32),
            pltpu.VMEM((1, G), jnp.int32),
            pltpu.VMEM((1, G), jnp.int32),
            pltpu.VMEM((16,), jnp.float32),
            pltpu.VMEM((16,), jnp.float32),
            pltpu.VMEM_SHARED((ACC, D), jnp.float32),
            pltpu.SemaphoreType.DMA,
            pltpu.SemaphoreType.DMA,
            pltpu.SemaphoreType.DMA,
            pltpu.SemaphoreType.DMA,
            pltpu.SemaphoreType.DMA,
        ],
    )
    return fn(s, esp, edp, esc, edc, esr, edr, hp, hc, emb_pad,
              tabs_flat, stats_flat)


def _fin_body(nums_ref, dens_ref, embs_ref, qslot_ref, wl_ref, bl_ref,
              wr_ref, bp_ref, bc_ref, out_ref, res_ref):
    num = nums_ref[...]
    den = dens_ref[...]
    n_p = num[0, 0, :B] + num[1, 0, :B]
    n_c = num[0, 1, :B] + num[1, 1, :B]
    n_r = num[0, 2, :B] + num[1, 2, :B]
    d_p = jnp.sum(den[0], axis=0)[:B]
    d_c = jnp.sum(den[1], axis=0)[:B]
    d_r = jnp.sum(den[2], axis=0)[:B]
    o1 = n_p / (d_p + 1e-16)[:, None] + bp_ref[...]
    o2 = n_c / (d_c + 1e-16)[:, None] + bc_ref[...]
    mean = n_r / jnp.maximum(d_r, 1.0)[:, None]
    o3 = (jnp.dot(mean, wl_ref[...], preferred_element_type=jnp.float32)
          + bl_ref[...]
          + jnp.dot(embs_ref[...], wr_ref[...],
                    preferred_element_type=jnp.float32))
    res_ref[...] = (o1 + o2 + o3) * jnp.float32(1.0 / 3.0)

    def _gather(b, carry):
        idx = qslot_ref[0, b]
        out_ref[0, pl.ds(b, 1), :] = res_ref[pl.ds(idx, 1), :]
        return carry
    lax.fori_loop(0, B, _gather, 0)


def _finish(nums, dens, embs, qslot, W_l, b_l, W_r, b_p, b_c):
    return pl.pallas_call(
        _fin_body,
        grid=(T,),
        in_specs=[
            pl.BlockSpec((2, 3, ACC, D), lambda t: (0, t, 0, 0)),
            pl.BlockSpec((3, NW, DACC), lambda t: (t, 0, 0)),
            pl.BlockSpec((B, D), lambda t: (0, 0)),
            pl.BlockSpec((1, B), lambda t: (0, 0), memory_space=pltpu.SMEM),
            pl.BlockSpec((D, D), lambda t: (0, 0)),
            pl.BlockSpec((1, D), lambda t: (0, 0)),
            pl.BlockSpec((D, D), lambda t: (0, 0)),
            pl.BlockSpec((1, D), lambda t: (0, 0)),
            pl.BlockSpec((1, D), lambda t: (0, 0)),
        ],
        out_specs=pl.BlockSpec((1, B, D), lambda t: (t, 0, 0)),
        out_shape=jax.ShapeDtypeStruct((T, B, D), jnp.float32),
        scratch_shapes=[pltpu.VMEM((B, D), jnp.float32)],
    )(nums, dens.reshape(6, NW, DACC), embs, qslot.reshape(1, B),
      W_l, b_l.reshape(1, D), W_r, b_p.reshape(1, D), b_c.reshape(1, D))


def kernel(s, t_s, t_e, emb, W_p, att_src_p, att_dst_p, b_p,
           W_c, att_src_c, att_dst_c, b_c, W_l, b_l, W_r,
           ei_parent, ei_child, ei_relate):
    del t_s, t_e
    emb_pad = jnp.pad(emb, ((0, NP - N), (0, 0)))
    s = s.astype(jnp.int32)
    ei_p = ei_parent.astype(jnp.int32)
    ei_c = ei_child.astype(jnp.int32)
    ei_r = ei_relate.astype(jnp.int32)
    hp, hc, tabs, stats = _prep(emb_pad, W_p, att_src_p, att_dst_p,
                                W_c, att_src_c, att_dst_c)
    nums, dens, qslot, embs = _sc_edges(s, ei_p, ei_c, ei_r,
                                        hp, hc, emb_pad, tabs, stats)
    return _finish(nums, dens, embs, qslot, W_l, b_l, W_r, b_p, b_c)
